# baseline (device time: 64582 ns/iter reference)
import jax
import jax.numpy as jnp
from jax import lax
from jax.experimental import pallas as pl
from jax.experimental.pallas import tpu as pltpu

N_DEV = 16
HQ_PER = 8
DH = 128
SQ = 256
NQB = 4
QBS = 64
NCPB = SQ // QBS
CHUNK = SQ // N_DEV
DM = 1024
SCALE = 0.08838834764831843
BF16 = jnp.bfloat16
F32 = jnp.float32


def kernel(x, Wq, K_ext, V_ext, Wo):
    K5 = K_ext.reshape(16, NQB, QBS, 128, DH)
    V5 = V_ext.reshape(16, NQB, QBS, 128, DH)

    def body(x_ref, wq_ref, k_ref, v_ref, wo_ref, out_ref,
             kbuf, vbuf, pbuf, recv_buf,
             kv_sems, send1, recv1, send2, recv2):
        me = lax.axis_index("i")
        h0 = me * HQ_PER

        def start_kv(qb, slot):
            for h in range(HQ_PER):
                pltpu.make_async_copy(
                    k_ref.at[:, qb, :, h0 + h, :],
                    kbuf.at[slot, h], kv_sems.at[slot, 0]).start()
                pltpu.make_async_copy(
                    v_ref.at[:, qb, :, h0 + h, :],
                    vbuf.at[slot, h], kv_sems.at[slot, 1]).start()

        def wait_kv(qb, slot):
            for h in range(HQ_PER):
                pltpu.make_async_copy(
                    k_ref.at[:, qb, :, h0 + h, :],
                    kbuf.at[slot, h], kv_sems.at[slot, 0]).wait()
                pltpu.make_async_copy(
                    v_ref.at[:, qb, :, h0 + h, :],
                    vbuf.at[slot, h], kv_sems.at[slot, 1]).wait()

        def reduce_and_broadcast():
            for o in range(1, N_DEV):
                d = (me + o) % N_DEV
                pltpu.make_async_remote_copy(
                    src_ref=pbuf.at[pl.ds(0, CHUNK), :],
                    dst_ref=recv_buf.at[d],
                    send_sem=send1.at[d],
                    recv_sem=recv1.at[d],
                    device_id=(d,),
                    device_id_type=pl.DeviceIdType.MESH,
                ).wait_recv()
            own = pbuf[pl.ds(me * CHUNK, CHUNK), :]
            idx = jax.lax.broadcasted_iota(jnp.int32, (N_DEV, 1, 1), 0)
            red = own + jnp.sum(
                jnp.where(idx != me, recv_buf[...], 0.0), axis=0)
            out_ref[0, pl.ds(me * CHUNK, CHUNK), :] = red
            for o in range(1, N_DEV):
                d = (me + o) % N_DEV
                pltpu.make_async_remote_copy(
                    src_ref=out_ref.at[0, pl.ds(me * CHUNK, CHUNK), :],
                    dst_ref=out_ref.at[0, pl.ds(me * CHUNK, CHUNK), :],
                    send_sem=send2.at[d],
                    recv_sem=recv2.at[me],
                    device_id=(d,),
                    device_id_type=pl.DeviceIdType.MESH,
                ).start()

        start_kv(0, 0)

        Qb = jnp.dot(x_ref[0].astype(BF16), wq_ref[...].astype(BF16),
                     preferred_element_type=F32).astype(BF16)
        wob = wo_ref[...].astype(BF16)

        for qb in range(NQB):
            slot = qb % 2
            if qb + 1 < NQB:
                start_kv(qb + 1, (qb + 1) % 2)
            wait_kv(qb, slot)
            ctx_heads = []
            for h in range(HQ_PER):
                q = Qb[qb * QBS:(qb + 1) * QBS, h * DH:(h + 1) * DH]
                k = kbuf[slot, h].reshape(16 * QBS, DH).astype(BF16)
                v = vbuf[slot, h].reshape(16 * QBS, DH).astype(BF16)
                s = jax.lax.dot_general(
                    q, k, (((1,), (1,)), ((), ())),
                    preferred_element_type=F32) * SCALE
                w = jnp.exp(s)
                ctx = jnp.dot(w.astype(BF16), v, preferred_element_type=F32)
                ctx_heads.append(ctx / jnp.sum(w, axis=-1, keepdims=True))
            ctx_qb = jnp.concatenate(ctx_heads, axis=1).astype(BF16)
            pbuf[qb * QBS:(qb + 1) * QBS, :] = jnp.dot(
                ctx_qb, wob, preferred_element_type=F32)

            for j in range(NCPB):
                c = NCPB * qb + (me + j) % NCPB

                @pl.when(me != c)
                def _(c=c):
                    pltpu.make_async_remote_copy(
                        src_ref=pbuf.at[pl.ds(c * CHUNK, CHUNK), :],
                        dst_ref=recv_buf.at[me],
                        send_sem=send1.at[c],
                        recv_sem=recv1.at[me],
                        device_id=(c,),
                        device_id_type=pl.DeviceIdType.MESH,
                    ).start()

            if qb >= 1:
                @pl.when(me // NCPB == qb - 1)
                def _():
                    reduce_and_broadcast()

        @pl.when(me // NCPB == NQB - 1)
        def _():
            reduce_and_broadcast()

        for o in range(1, N_DEV):
            d = (me + o) % N_DEV
            pltpu.make_async_remote_copy(
                src_ref=out_ref.at[0, pl.ds(d * CHUNK, CHUNK), :],
                dst_ref=out_ref.at[0, pl.ds(d * CHUNK, CHUNK), :],
                send_sem=send2.at[d],
                recv_sem=recv2.at[d],
                device_id=(d,),
                device_id_type=pl.DeviceIdType.MESH,
            ).wait_recv()

        for c in range(N_DEV):
            @pl.when(me != c)
            def _(c=c):
                pltpu.make_async_remote_copy(
                    src_ref=pbuf.at[pl.ds(c * CHUNK, CHUNK), :],
                    dst_ref=recv_buf.at[me],
                    send_sem=send1.at[c],
                    recv_sem=recv1.at[me],
                    device_id=(c,),
                    device_id_type=pl.DeviceIdType.MESH,
                ).wait_send()
        for o in range(1, N_DEV):
            d = (me + o) % N_DEV
            pltpu.make_async_remote_copy(
                src_ref=out_ref.at[0, pl.ds(me * CHUNK, CHUNK), :],
                dst_ref=out_ref.at[0, pl.ds(me * CHUNK, CHUNK), :],
                send_sem=send2.at[d],
                recv_sem=recv2.at[me],
                device_id=(d,),
                device_id_type=pl.DeviceIdType.MESH,
            ).wait_send()

    return pl.pallas_call(
        body,
        out_shape=jax.ShapeDtypeStruct((1, SQ, DM), jnp.float32),
        in_specs=[
            pl.BlockSpec(memory_space=pltpu.VMEM),
            pl.BlockSpec(memory_space=pltpu.VMEM),
            pl.BlockSpec(memory_space=pl.ANY),
            pl.BlockSpec(memory_space=pl.ANY),
            pl.BlockSpec(memory_space=pltpu.VMEM),
        ],
        out_specs=pl.BlockSpec(memory_space=pltpu.VMEM),
        scratch_shapes=[
            pltpu.VMEM((2, HQ_PER, 16, QBS, DH), jnp.float32),
            pltpu.VMEM((2, HQ_PER, 16, QBS, DH), jnp.float32),
            pltpu.VMEM((SQ, DM), jnp.float32),
            pltpu.VMEM((N_DEV, CHUNK, DM), jnp.float32),
            pltpu.SemaphoreType.DMA((2, 2)),
            pltpu.SemaphoreType.DMA((N_DEV,)),
            pltpu.SemaphoreType.DMA((N_DEV,)),
            pltpu.SemaphoreType.DMA((N_DEV,)),
            pltpu.SemaphoreType.DMA((N_DEV,)),
        ],
    )(x, Wq, K5, V5, Wo)


# device time: 43738 ns/iter; 1.4766x vs baseline; 1.4766x over previous
import jax
import jax.numpy as jnp
from jax import lax
from jax.experimental import pallas as pl
from jax.experimental.pallas import tpu as pltpu

N_DEV = 16
HQ_PER = 8
DH = 128
SQ = 256
NQB = 4
QBS = 64
NCPB = SQ // QBS
CHUNK = SQ // N_DEV
DM = 1024
SCALE = 0.08838834764831843
BF16 = jnp.bfloat16
F32 = jnp.float32


def kernel(x, Wq, K_ext, V_ext, Wo):
    K5 = K_ext.reshape(16, NQB, QBS, 128, DH)
    V5 = V_ext.reshape(16, NQB, QBS, 128, DH)

    def body(x_ref, wq_ref, k_ref, v_ref, wo_ref, out_ref,
             kbuf, vbuf, pbuf, recv_buf, gbuf,
             kv_sems, send1, recv1, send2, recv2):
        me = lax.axis_index("i")
        h0 = me * HQ_PER

        def start_kv(qb, slot):
            for h in range(HQ_PER):
                pltpu.make_async_copy(
                    k_ref.at[:, qb, :, h0 + h, :],
                    kbuf.at[slot, h], kv_sems.at[slot, 0]).start()
                pltpu.make_async_copy(
                    v_ref.at[:, qb, :, h0 + h, :],
                    vbuf.at[slot, h], kv_sems.at[slot, 1]).start()

        def wait_kv(qb, slot):
            for h in range(HQ_PER):
                pltpu.make_async_copy(
                    k_ref.at[:, qb, :, h0 + h, :],
                    kbuf.at[slot, h], kv_sems.at[slot, 0]).wait()
                pltpu.make_async_copy(
                    v_ref.at[:, qb, :, h0 + h, :],
                    vbuf.at[slot, h], kv_sems.at[slot, 1]).wait()

        start_kv(0, 0)

        Qb = jnp.dot(x_ref[0].astype(BF16), wq_ref[...].astype(BF16),
                     preferred_element_type=F32).astype(BF16)
        wob = wo_ref[...].astype(BF16)

        for qb in range(NQB):
            slot = qb % 2
            if qb + 1 < NQB:
                start_kv(qb + 1, (qb + 1) % 2)
            wait_kv(qb, slot)
            ctx_heads = []
            for h in range(HQ_PER):
                q = Qb[qb * QBS:(qb + 1) * QBS, h * DH:(h + 1) * DH]
                k = kbuf[slot, h].reshape(16 * QBS, DH).astype(BF16)
                v = vbuf[slot, h].reshape(16 * QBS, DH).astype(BF16)
                s = jax.lax.dot_general(
                    q, k, (((1,), (1,)), ((), ())),
                    preferred_element_type=F32) * SCALE
                w = jnp.exp(s)
                ctx = jnp.dot(w.astype(BF16), v, preferred_element_type=F32)
                ctx_heads.append(ctx / jnp.sum(w, axis=-1, keepdims=True))
            ctx_qb = jnp.concatenate(ctx_heads, axis=1).astype(BF16)
            pbuf[qb * QBS:(qb + 1) * QBS, :] = jnp.dot(
                ctx_qb, wob, preferred_element_type=F32).astype(BF16)

            for j in range(NCPB):
                c = NCPB * qb + (me + j) % NCPB

                @pl.when(me != c)
                def _(c=c):
                    pltpu.make_async_remote_copy(
                        src_ref=pbuf.at[pl.ds(c * CHUNK, CHUNK), :],
                        dst_ref=recv_buf.at[me],
                        send_sem=send1.at[c],
                        recv_sem=recv1.at[me],
                        device_id=(c,),
                        device_id_type=pl.DeviceIdType.MESH,
                    ).start()

        for o in range(1, N_DEV):
            d = (me + o) % N_DEV
            pltpu.make_async_remote_copy(
                src_ref=pbuf.at[pl.ds(0, CHUNK), :],
                dst_ref=recv_buf.at[d],
                send_sem=send1.at[d],
                recv_sem=recv1.at[d],
                device_id=(d,),
                device_id_type=pl.DeviceIdType.MESH,
            ).wait_recv()
        own = pbuf[pl.ds(me * CHUNK, CHUNK), :].astype(F32)
        idx = jax.lax.broadcasted_iota(jnp.int32, (N_DEV, 1, 1), 0)
        red = own + jnp.sum(
            jnp.where(idx != me, recv_buf[...].astype(F32), 0.0), axis=0)
        gbuf[pl.ds(me * CHUNK, CHUNK), :] = red.astype(BF16)

        for o in range(1, N_DEV):
            d = (me + o) % N_DEV
            pltpu.make_async_remote_copy(
                src_ref=gbuf.at[pl.ds(me * CHUNK, CHUNK), :],
                dst_ref=gbuf.at[pl.ds(me * CHUNK, CHUNK), :],
                send_sem=send2.at[d],
                recv_sem=recv2.at[me],
                device_id=(d,),
                device_id_type=pl.DeviceIdType.MESH,
            ).start()

        for o in range(1, N_DEV):
            d = (me + o) % N_DEV
            pltpu.make_async_remote_copy(
                src_ref=gbuf.at[pl.ds(d * CHUNK, CHUNK), :],
                dst_ref=gbuf.at[pl.ds(d * CHUNK, CHUNK), :],
                send_sem=send2.at[d],
                recv_sem=recv2.at[d],
                device_id=(d,),
                device_id_type=pl.DeviceIdType.MESH,
            ).wait_recv()

        out_ref[0, :, :] = gbuf[...].astype(F32)

        for c in range(N_DEV):
            @pl.when(me != c)
            def _(c=c):
                pltpu.make_async_remote_copy(
                    src_ref=pbuf.at[pl.ds(c * CHUNK, CHUNK), :],
                    dst_ref=recv_buf.at[me],
                    send_sem=send1.at[c],
                    recv_sem=recv1.at[me],
                    device_id=(c,),
                    device_id_type=pl.DeviceIdType.MESH,
                ).wait_send()
        for o in range(1, N_DEV):
            d = (me + o) % N_DEV
            pltpu.make_async_remote_copy(
                src_ref=gbuf.at[pl.ds(me * CHUNK, CHUNK), :],
                dst_ref=gbuf.at[pl.ds(me * CHUNK, CHUNK), :],
                send_sem=send2.at[d],
                recv_sem=recv2.at[me],
                device_id=(d,),
                device_id_type=pl.DeviceIdType.MESH,
            ).wait_send()

    return pl.pallas_call(
        body,
        out_shape=jax.ShapeDtypeStruct((1, SQ, DM), jnp.float32),
        in_specs=[
            pl.BlockSpec(memory_space=pltpu.VMEM),
            pl.BlockSpec(memory_space=pltpu.VMEM),
            pl.BlockSpec(memory_space=pl.ANY),
            pl.BlockSpec(memory_space=pl.ANY),
            pl.BlockSpec(memory_space=pltpu.VMEM),
        ],
        out_specs=pl.BlockSpec(memory_space=pltpu.VMEM),
        scratch_shapes=[
            pltpu.VMEM((2, HQ_PER, 16, QBS, DH), jnp.float32),
            pltpu.VMEM((2, HQ_PER, 16, QBS, DH), jnp.float32),
            pltpu.VMEM((SQ, DM), BF16),
            pltpu.VMEM((N_DEV, CHUNK, DM), BF16),
            pltpu.VMEM((SQ, DM), BF16),
            pltpu.SemaphoreType.DMA((2, 2)),
            pltpu.SemaphoreType.DMA((N_DEV,)),
            pltpu.SemaphoreType.DMA((N_DEV,)),
            pltpu.SemaphoreType.DMA((N_DEV,)),
            pltpu.SemaphoreType.DMA((N_DEV,)),
        ],
    )(x, Wq, K5, V5, Wo)


# device time: 43274 ns/iter; 1.4924x vs baseline; 1.0107x over previous
import jax
import jax.numpy as jnp
from jax import lax
from jax.experimental import pallas as pl
from jax.experimental.pallas import tpu as pltpu

N_DEV = 16
HQ_PER = 8
DH = 128
SQ = 256
NQB = 4
QBS = 64
NCPB = SQ // QBS
CHUNK = SQ // N_DEV
DM = 1024
SCALE = 0.08838834764831843
BF16 = jnp.bfloat16
F32 = jnp.float32


def kernel(x, Wq, K_ext, V_ext, Wo):
    K5 = K_ext.reshape(16, NQB, QBS, 128, DH)
    V5 = V_ext.reshape(16, NQB, QBS, 128, DH)

    def body(x_ref, wq_ref, k_ref, v_ref, wo_ref, out_ref,
             kbuf, vbuf, pbuf, recv_buf, gbuf,
             kv_sems, send1, recv1, send2, recv2):
        me = lax.axis_index("i")
        h0 = me * HQ_PER

        def start_kv(qb, slot):
            for h in range(HQ_PER):
                pltpu.make_async_copy(
                    k_ref.at[:, qb, :, h0 + h, :],
                    kbuf.at[slot, h], kv_sems.at[slot, 0]).start()
                pltpu.make_async_copy(
                    v_ref.at[:, qb, :, h0 + h, :],
                    vbuf.at[slot, h], kv_sems.at[slot, 1]).start()

        def wait_kv(qb, slot):
            for h in range(HQ_PER):
                pltpu.make_async_copy(
                    k_ref.at[:, qb, :, h0 + h, :],
                    kbuf.at[slot, h], kv_sems.at[slot, 0]).wait()
                pltpu.make_async_copy(
                    v_ref.at[:, qb, :, h0 + h, :],
                    vbuf.at[slot, h], kv_sems.at[slot, 1]).wait()

        start_kv(0, 0)

        Qb = jnp.dot(x_ref[0].astype(BF16), wq_ref[...].astype(BF16),
                     preferred_element_type=F32).astype(BF16)
        wob = wo_ref[...].astype(BF16)

        for qb in range(NQB):
            slot = qb % 2
            if qb + 1 < NQB:
                start_kv(qb + 1, (qb + 1) % 2)
            wait_kv(qb, slot)
            ctx_heads = []
            for h in range(HQ_PER):
                q = Qb[qb * QBS:(qb + 1) * QBS, h * DH:(h + 1) * DH]
                k = kbuf[slot, h].reshape(16 * QBS, DH).astype(BF16)
                v = vbuf[slot, h].reshape(16 * QBS, DH).astype(BF16)
                s = jax.lax.dot_general(
                    q, k, (((1,), (1,)), ((), ())),
                    preferred_element_type=F32) * SCALE
                w = jnp.exp(s)
                ctx = jnp.dot(w.astype(BF16), v, preferred_element_type=F32)
                ctx_heads.append(ctx / jnp.sum(w, axis=-1, keepdims=True))
            ctx_qb = jnp.concatenate(ctx_heads, axis=1).astype(BF16)
            pbuf[qb * QBS:(qb + 1) * QBS, :] = jnp.dot(
                ctx_qb, wob, preferred_element_type=F32).astype(BF16)

            for j in range(NCPB):
                c = NCPB * qb + (me + j) % NCPB

                @pl.when(me != c)
                def _(c=c):
                    pltpu.make_async_remote_copy(
                        src_ref=pbuf.at[pl.ds(c * CHUNK, CHUNK), :],
                        dst_ref=recv_buf.at[me],
                        send_sem=send1.at[c],
                        recv_sem=recv1.at[me],
                        device_id=(c,),
                        device_id_type=pl.DeviceIdType.MESH,
                    ).start()

        red = pbuf[pl.ds(me * CHUNK, CHUNK), :].astype(F32)
        for o in range(1, N_DEV):
            d = (me + o) % N_DEV
            pltpu.make_async_remote_copy(
                src_ref=pbuf.at[pl.ds(0, CHUNK), :],
                dst_ref=recv_buf.at[d],
                send_sem=send1.at[d],
                recv_sem=recv1.at[d],
                device_id=(d,),
                device_id_type=pl.DeviceIdType.MESH,
            ).wait_recv()
            red = red + recv_buf[d].astype(F32)
        gbuf[pl.ds(me * CHUNK, CHUNK), :] = red.astype(BF16)

        for o in range(1, N_DEV):
            d = (me + o) % N_DEV
            pltpu.make_async_remote_copy(
                src_ref=gbuf.at[pl.ds(me * CHUNK, CHUNK), :],
                dst_ref=gbuf.at[pl.ds(me * CHUNK, CHUNK), :],
                send_sem=send2.at[d],
                recv_sem=recv2.at[me],
                device_id=(d,),
                device_id_type=pl.DeviceIdType.MESH,
            ).start()

        for o in range(1, N_DEV):
            d = (me + o) % N_DEV
            pltpu.make_async_remote_copy(
                src_ref=gbuf.at[pl.ds(d * CHUNK, CHUNK), :],
                dst_ref=gbuf.at[pl.ds(d * CHUNK, CHUNK), :],
                send_sem=send2.at[d],
                recv_sem=recv2.at[d],
                device_id=(d,),
                device_id_type=pl.DeviceIdType.MESH,
            ).wait_recv()

        out_ref[0, :, :] = gbuf[...].astype(F32)

        for c in range(N_DEV):
            @pl.when(me != c)
            def _(c=c):
                pltpu.make_async_remote_copy(
                    src_ref=pbuf.at[pl.ds(c * CHUNK, CHUNK), :],
                    dst_ref=recv_buf.at[me],
                    send_sem=send1.at[c],
                    recv_sem=recv1.at[me],
                    device_id=(c,),
                    device_id_type=pl.DeviceIdType.MESH,
                ).wait_send()
        for o in range(1, N_DEV):
            d = (me + o) % N_DEV
            pltpu.make_async_remote_copy(
                src_ref=gbuf.at[pl.ds(me * CHUNK, CHUNK), :],
                dst_ref=gbuf.at[pl.ds(me * CHUNK, CHUNK), :],
                send_sem=send2.at[d],
                recv_sem=recv2.at[me],
                device_id=(d,),
                device_id_type=pl.DeviceIdType.MESH,
            ).wait_send()

    return pl.pallas_call(
        body,
        out_shape=jax.ShapeDtypeStruct((1, SQ, DM), jnp.float32),
        in_specs=[
            pl.BlockSpec(memory_space=pltpu.VMEM),
            pl.BlockSpec(memory_space=pltpu.VMEM),
            pl.BlockSpec(memory_space=pl.ANY),
            pl.BlockSpec(memory_space=pl.ANY),
            pl.BlockSpec(memory_space=pltpu.VMEM),
        ],
        out_specs=pl.BlockSpec(memory_space=pltpu.VMEM),
        scratch_shapes=[
            pltpu.VMEM((2, HQ_PER, 16, QBS, DH), jnp.float32),
            pltpu.VMEM((2, HQ_PER, 16, QBS, DH), jnp.float32),
            pltpu.VMEM((SQ, DM), BF16),
            pltpu.VMEM((N_DEV, CHUNK, DM), BF16),
            pltpu.VMEM((SQ, DM), BF16),
            pltpu.SemaphoreType.DMA((2, 2)),
            pltpu.SemaphoreType.DMA((N_DEV,)),
            pltpu.SemaphoreType.DMA((N_DEV,)),
            pltpu.SemaphoreType.DMA((N_DEV,)),
            pltpu.SemaphoreType.DMA((N_DEV,)),
        ],
    )(x, Wq, K5, V5, Wo)
